# Initial kernel scaffold; baseline (speedup 1.0000x reference)
import jax
import jax.numpy as jnp
from jax.experimental import pallas as pl


def kernel(x, gate_W, gate_b, expert_W, expert_b):
    raise NotImplementedError("write your pallas kernel here")


# speedup vs baseline: 7.6473x; 7.6473x over previous
import functools

import jax
import jax.numpy as jnp
from jax.experimental import pallas as pl
import jax.experimental.pallas.tpu as pltpu

HIDDEN = 1024
E = 8
T_TOTAL = 8192
TB = 512
NUM_TB = T_TOTAL // TB


def _dot_t(a, b):
    return jax.lax.dot_general(
        a, b, (((1,), (1,)), ((), ())), preferred_element_type=jnp.float32
    )


def _moe_kernel(x_ref, x8_ref, gw_ref, gb_ref, ew_ref, eb_ref,
                out_ref, logits_ref, acc_ref):
    i = pl.program_id(0)

    xb = x_ref[:, :]
    gw = gw_ref[:, :]
    gb = gb_ref[:, :]
    logits_ref[:, :] = _dot_t(xb, gw) + gb

    out_ref[:, :] = jnp.zeros((TB, HIDDEN), jnp.float32)

    @pl.when(i < E)
    def _expert_step():
        x8 = x8_ref[:, :]
        l8 = _dot_t(x8, gw) + gb
        m = jnp.max(l8, axis=-1, keepdims=True)
        p = jnp.exp(l8 - m)
        p = p / jnp.sum(p, axis=-1, keepdims=True)

        iota_e = jax.lax.broadcasted_iota(jnp.int32, (E, E), 1).astype(jnp.float32)
        iota_r = jax.lax.broadcasted_iota(jnp.int32, (E, E), 0).astype(jnp.float32)

        rank = jnp.zeros((E, E), jnp.float32)
        for e2 in range(E):
            pe2 = p[:, e2:e2 + 1]
            rank = rank + jnp.where(
                (pe2 > p) | ((pe2 == p) & (e2 < iota_e)), 1.0, 0.0)

        sp = jnp.zeros((E, E), jnp.float32)
        for e in range(E):
            sp = sp + jnp.where(rank[:, e:e + 1] == iota_e,
                                p[:, e:e + 1], 0.0)
        sp = sp / jnp.sum(sp, axis=-1, keepdims=True)

        fi = i.astype(jnp.float32)
        rank_i = jnp.sum(jnp.where(iota_r == fi, rank, 0.0),
                         axis=0, keepdims=True)
        c = sp * jnp.where(rank_i == iota_r, 1.0, 0.0)

        y = _dot_t(x8, ew_ref[0]) + eb_ref[0]

        @pl.when(i == 0)
        def _init():
            acc_ref[:, :] = jnp.zeros((E, HIDDEN), jnp.float32)

        acc_ref[:, :] += jax.lax.dot_general(
            c, y, (((1,), (0,)), ((), ())),
            preferred_element_type=jnp.float32)

    @pl.when(i == NUM_TB - 1)
    def _final():
        out_ref[0:E, :] = acc_ref[:, :]


@jax.jit
def kernel(x, gate_W, gate_b, expert_W, expert_b):
    B, S, H = x.shape
    xf = x.reshape(B * S, H)
    gb2 = gate_b.reshape(1, E)
    eb3 = expert_b.reshape(E, 1, H)

    grid = (NUM_TB,)
    out, logits = pl.pallas_call(
        _moe_kernel,
        grid=grid,
        in_specs=[
            pl.BlockSpec((TB, H), lambda i: ((i + 1) % NUM_TB, 0)),
            pl.BlockSpec((E, H), lambda i: (0, 0)),
            pl.BlockSpec((E, H), lambda i: (0, 0)),
            pl.BlockSpec((1, E), lambda i: (0, 0)),
            pl.BlockSpec((1, H, H), lambda i: (jnp.minimum(i, E - 1), 0, 0)),
            pl.BlockSpec((1, 1, H), lambda i: (jnp.minimum(i, E - 1), 0, 0)),
        ],
        out_specs=[
            pl.BlockSpec((TB, H), lambda i: ((i + 1) % NUM_TB, 0)),
            pl.BlockSpec((TB, E), lambda i: ((i + 1) % NUM_TB, 0)),
        ],
        out_shape=[
            jax.ShapeDtypeStruct((B * S, H), jnp.float32),
            jax.ShapeDtypeStruct((B * S, E), jnp.float32),
        ],
        scratch_shapes=[pltpu.VMEM((E, HIDDEN), jnp.float32)],
    )(xf, xf, gate_W, gb2, expert_W, eb3)

    return out.reshape(B, S, H), logits
